# baseline (speedup 1.0000x reference)
import jax
import jax.numpy as jnp
from jax import lax
from jax.experimental import pallas as pl
from jax.experimental.pallas import tpu as pltpu
from jax.experimental.pallas import tpu_sc as plsc

U = 100000
I = 100000
D = 16
TAU = 0.2
LMBD_SSL = 0.1
LMBD_REG = 1e-4
DROP = 0.1
N = U + I

NC = 2
NS = 16
BLK = 128
QD = 8
GE = BLK * QD
SPAN = 6256
ACC_ROWS = NS * SPAN
TRASH = U
NP = NC * ACC_ROWS


SB = 4
SBE = SB * BLK


def _spmm_body(t_hbm, rows_hbm, cols_hbm, bnd_hbm, out_hbm,
               acc_sh, bvec, rbufA, cbufA, libufA, rbufB, cbufB, libufB,
               gbuf0, gbuf1, tibuf,
               sem_ia, sem_ib, sem_g0, sem_g1, sem_s0, sem_s1):
    c = lax.axis_index("c")
    s = lax.axis_index("s")
    lanes = lax.iota(jnp.int32, 16)
    zero16 = jnp.zeros((16,), jnp.float32)

    def zfill(i, carry):
        gbuf0[i, :] = zero16
        return carry

    lax.fori_loop(0, SBE, zfill, 0)
    for r in range(SPAN // SBE):
        pltpu.sync_copy(
            gbuf0, acc_sh.at[pl.ds(pl.multiple_of(s * SPAN + r * SBE, 8), SBE)])
    rem = SPAN % SBE
    pltpu.sync_copy(
        gbuf0.at[pl.ds(0, rem)],
        acc_sh.at[pl.ds(pl.multiple_of(s * SPAN + (SPAN // SBE) * SBE, 8), rem)])
    for p in range(BLK // 16):
        tibuf[0, pl.ds(p * 16, 16)] = jnp.full((16,), TRASH, jnp.int32)
    plsc.subcore_barrier()

    pltpu.sync_copy(bnd_hbm, bvec)
    fs = c * 16 + s

    def pick(base):
        acc = jnp.int32(0)
        for k in range(2):
            chunk = bvec[pl.ds(base + k * 16, 16)]
            acc = acc + jnp.sum(jnp.where((k * 16) + lanes == fs, chunk, 0))
        return acc

    start = pick(0)
    end = pick(32)
    ngroups = (end - start + (GE - 1)) // GE
    npairs = (ngroups + 1) // 2
    rowbase = c * U

    def crow_of(g):
        return pl.multiple_of((start + g * GE) // BLK, 8)

    def ids_issue(g, rbuf, cbuf, sem):
        cr = crow_of(g)
        pltpu.async_copy(rows_hbm.at[pl.ds(cr, QD)], rbuf, sem)
        pltpu.async_copy(cols_hbm.at[pl.ds(cr, QD)], cbuf, sem)

    def ids_drain(g, rbuf, cbuf, sem):
        cr = crow_of(g)
        pltpu.make_async_copy(rows_hbm.at[pl.ds(cr, QD)], rbuf, sem).wait()
        pltpu.make_async_copy(cols_hbm.at[pl.ds(cr, QD)], cbuf, sem).wait()

    gsets = ((gbuf0, sem_g0, sem_s0), (gbuf1, sem_g1, sem_s1))

    def sdrain(h):
        gb, _, sem_s = gsets[h]
        for k in range(SB):
            pltpu.make_async_copy(gb.at[pl.ds(k * BLK, BLK)],
                                  acc_sh.at[tibuf.at[0]], sem_s).wait()

    def group(g, rbuf, cbuf, libuf):
        goff = start + g * GE
        for q in range(QD):
            for p in range(BLK // 16):
                rid = rbuf[q, pl.ds(p * 16, 16)]
                gidx = (goff + q * BLK + p * 16) + lanes
                local = rid - rowbase
                valid = (gidx < end) & (local >= 0) & (local < U)
                libuf[q, pl.ds(p * 16, 16)] = jnp.where(valid, local, TRASH)
        for h in range(2):
            gb, sem_g, sem_s = gsets[h]
            sdrain(h)
            gds = [pltpu.async_copy(t_hbm.at[cbuf.at[h * SB + k]],
                                    gb.at[pl.ds(k * BLK, BLK)], sem_g)
                   for k in range(SB)]
            for k in range(SB):
                gds[k].wait()
                pltpu.async_copy(gb.at[pl.ds(k * BLK, BLK)],
                                 acc_sh.at[libuf.at[h * SB + k]], sem_s,
                                 add=True)

    for h in range(2):
        gb, _, sem_s = gsets[h]
        for k in range(SB):
            pltpu.async_copy(gb.at[pl.ds(k * BLK, BLK)],
                             acc_sh.at[tibuf.at[0]], sem_s, add=True)
    ids_issue(0, rbufA, cbufA, sem_ia)

    def pair(p, carry):
        g0 = p * 2
        ids_drain(g0, rbufA, cbufA, sem_ia)
        ids_issue(g0 + 1, rbufB, cbufB, sem_ib)
        group(g0, rbufA, cbufA, libufA)
        ids_drain(g0 + 1, rbufB, cbufB, sem_ib)
        ids_issue(g0 + 2, rbufA, cbufA, sem_ia)
        group(g0 + 1, rbufB, cbufB, libufB)
        return carry

    lax.fori_loop(0, npairs, pair, 0)
    ids_drain(2 * npairs, rbufA, cbufA, sem_ia)
    sdrain(0)
    sdrain(1)

    plsc.subcore_barrier()
    src_off = pl.multiple_of(s * SPAN, 8)
    dst_off = pl.multiple_of(c * ACC_ROWS + s * SPAN, 8)
    pltpu.sync_copy(acc_sh.at[pl.ds(src_off, SPAN)],
                    out_hbm.at[pl.ds(dst_off, SPAN)])


def _make_spmm():
    mesh = plsc.VectorSubcoreMesh(core_axis_name="c", subcore_axis_name="s",
                                  num_cores=NC, num_subcores=NS)
    return pl.kernel(
        _spmm_body,
        out_type=jax.ShapeDtypeStruct((NP, D), jnp.float32),
        mesh=mesh,
        scratch_types=[
            pltpu.VMEM_SHARED((ACC_ROWS, D), jnp.float32),
            pltpu.VMEM((64,), jnp.int32),
            pltpu.VMEM((QD, BLK), jnp.int32),
            pltpu.VMEM((QD, BLK), jnp.int32),
            pltpu.VMEM((QD, BLK), jnp.int32),
            pltpu.VMEM((QD, BLK), jnp.int32),
            pltpu.VMEM((QD, BLK), jnp.int32),
            pltpu.VMEM((QD, BLK), jnp.int32),
            pltpu.VMEM((SBE, D), jnp.float32),
            pltpu.VMEM((SBE, D), jnp.float32),
            pltpu.VMEM((1, BLK), jnp.int32),
            pltpu.SemaphoreType.DMA,
            pltpu.SemaphoreType.DMA,
            pltpu.SemaphoreType.DMA,
            pltpu.SemaphoreType.DMA,
            pltpu.SemaphoreType.DMA,
            pltpu.SemaphoreType.DMA,
        ],
        compiler_params=pltpu.CompilerParams(use_tc_tiling_on_sc=False,
                                             needs_layout_passes=False),
    )


def _tile_bounds(lo, hi):
    sidx = jnp.arange(NS, dtype=jnp.int32)
    raw = lo + ((hi - lo) * sidx) // NS
    st = raw & ~jnp.int32(GE - 1)
    en = jnp.concatenate([st[1:], hi[None]])
    return st, en


def _edge_prep(rows, cols):
    e = rows.shape[0]
    lp = (e // GE + 4) * GE
    split = jnp.sum((rows < U).astype(jnp.int32))
    st0, en0 = _tile_bounds(jnp.int32(0), split)
    st1, en1 = _tile_bounds(split, jnp.int32(e))
    bnd = jnp.concatenate([st0, st1, en0, en1])
    rows_p = jnp.concatenate([rows, jnp.full((lp - e,), N, jnp.int32)])
    cols_adj = jnp.where(cols >= U, cols + (ACC_ROWS - U), cols)
    cols_p = jnp.concatenate([cols_adj, jnp.zeros((lp - e,), jnp.int32)])
    return (rows_p.reshape(lp // BLK, BLK), cols_p.reshape(lp // BLK, BLK),
            bnd)


B = 1024
GB = 4


def _bgather_body(tg_hbm, te_hbm, t1_hbm, t2_hbm, idx_hbm, out_hbm,
                  ibuf, gbuf, sem):
    c = lax.axis_index("c")
    s = lax.axis_index("s")
    w = c * 16 + s
    pltpu.sync_copy(idx_hbm.at[w], ibuf)

    def do(tbl):
        def _():
            gds = [pltpu.async_copy(tbl.at[ibuf.at[k]],
                                    gbuf.at[pl.ds(k * BLK, BLK)], sem)
                   for k in range(GB)]
            for g in gds:
                g.wait()
            pltpu.sync_copy(
                gbuf, out_hbm.at[pl.ds(pl.multiple_of(w * (GB * BLK), 8),
                                       GB * BLK)])
        return _

    pl.when((c == 0) & (s < 8))(do(tg_hbm))
    pl.when((c == 0) & (s >= 8))(do(te_hbm))
    pl.when((c == 1) & (s < 8))(do(t1_hbm))
    pl.when((c == 1) & (s >= 8))(do(t2_hbm))


def _make_bgather():
    mesh = plsc.VectorSubcoreMesh(core_axis_name="c", subcore_axis_name="s",
                                  num_cores=NC, num_subcores=NS)
    return pl.kernel(
        _bgather_body,
        out_type=jax.ShapeDtypeStruct((32 * GB * BLK, D), jnp.float32),
        mesh=mesh,
        scratch_types=[
            pltpu.VMEM((GB, BLK), jnp.int32),
            pltpu.VMEM((GB * BLK, D), jnp.float32),
            pltpu.SemaphoreType.DMA,
        ],
        compiler_params=pltpu.CompilerParams(use_tc_tiling_on_sc=False,
                                             needs_layout_passes=False),
    )


def _pack_sec(ids, nblk):
    a = ids.reshape(8, nblk, BLK)
    pad = jnp.zeros((8, GB - nblk, BLK), jnp.int32)
    return jnp.concatenate([a, pad], axis=1)


RBLK = 2944
NBLK_HALF = ACC_ROWS // RBLK


def _ssl_body(q_ref, t_ref, o_ref):
    b = pl.program_id(1)
    q = q_ref[0]
    tb = t_ref[...]
    s2 = jnp.sum(tb * tb, axis=1, keepdims=True)
    inv = 1.0 / jnp.maximum(jnp.sqrt(s2), 1e-12)
    sc = lax.dot_general(q, tb * inv, (((1,), (1,)), ((), ())),
                         preferred_element_type=jnp.float32)
    ridx = lax.broadcasted_iota(jnp.int32, (1, RBLK), 1) + b * RBLK
    z = jnp.where(ridx < U, jnp.exp(sc * (1.0 / TAU)), 0.0)
    r = jnp.sum(z, axis=1)

    @pl.when(b == 0)
    def _():
        o_ref[0, 0, :] = r

    @pl.when(b != 0)
    def _():
        o_ref[0, 0, :] = o_ref[0, 0, :] + r


def _ssl_sumexp(qn, tbl):
    out = pl.pallas_call(
        _ssl_body,
        grid=(2, NBLK_HALF),
        in_specs=[pl.BlockSpec((1, B, 16), lambda p, b: (p, 0, 0)),
                  pl.BlockSpec((RBLK, 16), lambda p, b: (p * NBLK_HALF + b, 0))],
        out_specs=pl.BlockSpec((1, 1, B), lambda p, b: (p, 0, 0)),
        out_shape=jax.ShapeDtypeStruct((2, 1, B), jnp.float32),
    )(qn, tbl)
    return out[:, 0, :]


def _normalize(x):
    return x / jnp.clip(jnp.linalg.norm(x, axis=1, keepdims=True), 1e-12, None)


def kernel(user_emb, item_emb, g_rows, g_cols, g_vals, g1_rows, g1_cols, g1_vals,
           g2_rows, g2_cols, g2_vals, user_id, item_id, neg_item_id):
    spmm = _make_spmm()
    bgather = _make_bgather()

    rp0, cp0, bnd0 = _edge_prep(g_rows, g_cols)
    rp1, cp1, bnd1 = _edge_prep(g1_rows, g1_cols)
    rp2, cp2, bnd2 = _edge_prep(g2_rows, g2_cols)

    deg_raw = spmm(jnp.ones((NP, D), jnp.float32), rp0, cp0, bnd0)[:, 0]
    deg = jnp.maximum(deg_raw, 1.0)
    invd = (1.0 / deg)[:, None]
    invd_drop = invd * (1.0 / (1.0 - DROP))

    all_emb = (jnp.zeros((NP, D), jnp.float32)
               .at[0:U].set(user_emb)
               .at[ACC_ROWS:ACC_ROWS + U].set(item_emb))
    t0 = all_emb * (deg ** -0.5)[:, None]

    def prop(rp, cp, bnd, scale):
        t1 = spmm(t0, rp, cp, bnd) * scale
        t2 = spmm(t1, rp, cp, bnd) * scale
        t3 = spmm(t2, rp, cp, bnd) * scale
        return t0 + t1 + t2 + t3

    light_g = jnp.sqrt(deg)[:, None] * prop(rp0, cp0, bnd0, invd) * 0.25
    tsum_1 = prop(rp1, cp1, bnd1, invd_drop)
    tsum_2 = prop(rp2, cp2, bnd2, invd_drop)

    iid = item_id + ACC_ROWS
    nid = neg_item_id + ACC_ROWS

    idx3d = jnp.concatenate([
        _pack_sec(jnp.concatenate([user_id, iid, nid]), 3),
        _pack_sec(jnp.concatenate([user_id, iid, nid]), 3),
        _pack_sec(jnp.concatenate([user_id, iid]), 2),
        _pack_sec(jnp.concatenate([user_id, iid]), 2),
    ], axis=0)
    rows = bgather(light_g, all_emb, tsum_1, tsum_2, idx3d)
    r4 = rows.reshape(32, GB, BLK, D)
    sec_g = r4[0:8, 0:3].reshape(3 * B, D)
    sec_e = r4[8:16, 0:3].reshape(3 * B, D)
    sec_1 = r4[16:24, 0:2].reshape(2 * B, D)
    sec_2 = r4[24:32, 0:2].reshape(2 * B, D)
    ue, pie, nie = sec_g[0:B], sec_g[B:2 * B], sec_g[2 * B:3 * B]
    ue_ego, pie_ego, nie_ego = sec_e[0:B], sec_e[B:2 * B], sec_e[2 * B:3 * B]

    pos_scores = jnp.sum(ue * pie, axis=1)
    neg_scores = jnp.sum(ue * nie, axis=1)
    bpr_loss = jnp.mean(jax.nn.softplus(neg_scores - pos_scores))
    reg_loss = (jnp.sum(ue_ego ** 2) + jnp.sum(pie_ego ** 2)
                + jnp.sum(nie_ego ** 2)) / (2.0 * B)

    ue1 = _normalize(sec_1[0:B])
    ie1 = _normalize(sec_1[B:2 * B])
    ue2 = _normalize(sec_2[0:B])
    ie2 = _normalize(sec_2[B:2 * B])
    pos_u = jnp.sum(ue1 * ue2, axis=1)
    pos_i = jnp.sum(ie1 * ie2, axis=1)

    qn = jnp.stack([ue1, ie1])
    zraw = _ssl_sumexp(qn, tsum_2)
    clog_u = jnp.log(zraw[0]) - pos_u / TAU
    clog_i = jnp.log(zraw[1]) - pos_i / TAU
    ssl_loss = jnp.sum(clog_u + clog_i)

    return bpr_loss + ssl_loss * LMBD_SSL + reg_loss * LMBD_REG


# speedup vs baseline: 1.2129x; 1.2129x over previous
import jax
import jax.numpy as jnp
from jax import lax
from jax.experimental import pallas as pl
from jax.experimental.pallas import tpu as pltpu
from jax.experimental.pallas import tpu_sc as plsc

U = 100000
I = 100000
D = 16
TAU = 0.2
LMBD_SSL = 0.1
LMBD_REG = 1e-4
DROP = 0.1
N = U + I

NC = 2
NS = 16
BLK = 128
QD = 8
GE = BLK * QD
SPAN = 6256
ACC_ROWS = NS * SPAN
TRASH = U
NP = NC * ACC_ROWS


SB = 4
SBE = SB * BLK


def _spmm_body(t_hbm, rows_hbm, cols_hbm, bnd_hbm, out_hbm,
               acc_sh, bvec, rbufA, cbufA, libufA, rbufB, cbufB, libufB,
               gbuf0, gbuf1, tibuf,
               sem_ia, sem_ib, sem_g0, sem_g1, sem_s0, sem_s1):
    c = lax.axis_index("c")
    s = lax.axis_index("s")
    lanes = lax.iota(jnp.int32, 16)
    zero16 = jnp.zeros((16,), jnp.float32)

    def zfill(i, carry):
        gbuf0[i, :] = zero16
        return carry

    lax.fori_loop(0, SBE, zfill, 0)
    for r in range(SPAN // SBE):
        pltpu.sync_copy(
            gbuf0, acc_sh.at[pl.ds(pl.multiple_of(s * SPAN + r * SBE, 8), SBE)])
    rem = SPAN % SBE
    pltpu.sync_copy(
        gbuf0.at[pl.ds(0, rem)],
        acc_sh.at[pl.ds(pl.multiple_of(s * SPAN + (SPAN // SBE) * SBE, 8), rem)])
    for p in range(BLK // 16):
        tibuf[0, pl.ds(p * 16, 16)] = jnp.full((16,), TRASH, jnp.int32)
    plsc.subcore_barrier()

    pltpu.sync_copy(bnd_hbm, bvec)
    fs = c * 16 + s

    def pick(base):
        acc = jnp.int32(0)
        for k in range(2):
            chunk = bvec[pl.ds(base + k * 16, 16)]
            acc = acc + jnp.sum(jnp.where((k * 16) + lanes == fs, chunk, 0))
        return acc

    start = pick(0)
    end = pick(32)
    ngroups = (end - start + (GE - 1)) // GE
    npairs = (ngroups + 1) // 2
    rowbase = c * U

    def crow_of(g):
        return pl.multiple_of((start + g * GE) // BLK, 8)

    def ids_issue(g, rbuf, cbuf, sem):
        cr = crow_of(g)
        pltpu.async_copy(rows_hbm.at[pl.ds(cr, QD)], rbuf, sem)
        pltpu.async_copy(cols_hbm.at[pl.ds(cr, QD)], cbuf, sem)

    def ids_drain(g, rbuf, cbuf, sem):
        cr = crow_of(g)
        pltpu.make_async_copy(rows_hbm.at[pl.ds(cr, QD)], rbuf, sem).wait()
        pltpu.make_async_copy(cols_hbm.at[pl.ds(cr, QD)], cbuf, sem).wait()

    def group(g, rbuf, cbuf, libuf):
        goff = start + g * GE
        for q in range(QD):
            for p in range(BLK // 16):
                rid = rbuf[q, pl.ds(p * 16, 16)]
                gidx = (goff + q * BLK + p * 16) + lanes
                local = rid - rowbase
                valid = (gidx < end) & (local >= 0) & (local < U)
                libuf[q, pl.ds(p * 16, 16)] = jnp.where(valid, local, TRASH)
        gds = [pltpu.async_copy(
                   t_hbm.at[cbuf.at[q]],
                   (gbuf0 if q < SB else gbuf1).at[pl.ds((q % SB) * BLK, BLK)],
                   sem_g0)
               for q in range(QD)]
        sds = []
        for q in range(QD):
            gds[q].wait()
            sds.append(pltpu.async_copy(
                (gbuf0 if q < SB else gbuf1).at[pl.ds((q % SB) * BLK, BLK)],
                acc_sh.at[libuf.at[q]], sem_s0, add=True))
        for d in sds:
            d.wait()

    ids_issue(0, rbufA, cbufA, sem_ia)

    def pair(p, carry):
        g0 = p * 2
        ids_drain(g0, rbufA, cbufA, sem_ia)
        ids_issue(g0 + 1, rbufB, cbufB, sem_ib)
        group(g0, rbufA, cbufA, libufA)
        ids_drain(g0 + 1, rbufB, cbufB, sem_ib)
        ids_issue(g0 + 2, rbufA, cbufA, sem_ia)
        group(g0 + 1, rbufB, cbufB, libufB)
        return carry

    lax.fori_loop(0, npairs, pair, 0)
    ids_drain(2 * npairs, rbufA, cbufA, sem_ia)

    plsc.subcore_barrier()
    src_off = pl.multiple_of(s * SPAN, 8)
    dst_off = pl.multiple_of(c * ACC_ROWS + s * SPAN, 8)
    pltpu.sync_copy(acc_sh.at[pl.ds(src_off, SPAN)],
                    out_hbm.at[pl.ds(dst_off, SPAN)])


def _make_spmm():
    mesh = plsc.VectorSubcoreMesh(core_axis_name="c", subcore_axis_name="s",
                                  num_cores=NC, num_subcores=NS)
    return pl.kernel(
        _spmm_body,
        out_type=jax.ShapeDtypeStruct((NP, D), jnp.float32),
        mesh=mesh,
        scratch_types=[
            pltpu.VMEM_SHARED((ACC_ROWS, D), jnp.float32),
            pltpu.VMEM((64,), jnp.int32),
            pltpu.VMEM((QD, BLK), jnp.int32),
            pltpu.VMEM((QD, BLK), jnp.int32),
            pltpu.VMEM((QD, BLK), jnp.int32),
            pltpu.VMEM((QD, BLK), jnp.int32),
            pltpu.VMEM((QD, BLK), jnp.int32),
            pltpu.VMEM((QD, BLK), jnp.int32),
            pltpu.VMEM((SBE, D), jnp.float32),
            pltpu.VMEM((SBE, D), jnp.float32),
            pltpu.VMEM((1, BLK), jnp.int32),
            pltpu.SemaphoreType.DMA,
            pltpu.SemaphoreType.DMA,
            pltpu.SemaphoreType.DMA,
            pltpu.SemaphoreType.DMA,
            pltpu.SemaphoreType.DMA,
            pltpu.SemaphoreType.DMA,
        ],
        compiler_params=pltpu.CompilerParams(use_tc_tiling_on_sc=False,
                                             needs_layout_passes=False),
    )


def _tile_bounds(lo, hi):
    sidx = jnp.arange(NS, dtype=jnp.int32)
    raw = lo + ((hi - lo) * sidx) // NS
    st = raw & ~jnp.int32(GE - 1)
    en = jnp.concatenate([st[1:], hi[None]])
    return st, en


def _edge_prep(rows, cols):
    e = rows.shape[0]
    lp = (e // GE + 4) * GE
    split = jnp.sum((rows < U).astype(jnp.int32))
    st0, en0 = _tile_bounds(jnp.int32(0), split)
    st1, en1 = _tile_bounds(split, jnp.int32(e))
    bnd = jnp.concatenate([st0, st1, en0, en1])
    rows_p = jnp.concatenate([rows, jnp.full((lp - e,), N, jnp.int32)])
    cols_adj = jnp.where(cols >= U, cols + (ACC_ROWS - U), cols)
    cols_p = jnp.concatenate([cols_adj, jnp.zeros((lp - e,), jnp.int32)])
    return (rows_p.reshape(lp // BLK, BLK), cols_p.reshape(lp // BLK, BLK),
            bnd)


B = 1024
GB = 4


def _bgather_body(tg_hbm, te_hbm, t1_hbm, t2_hbm, idx_hbm, out_hbm,
                  ibuf, gbuf, sem):
    c = lax.axis_index("c")
    s = lax.axis_index("s")
    w = c * 16 + s
    pltpu.sync_copy(idx_hbm.at[w], ibuf)

    def do(tbl):
        def _():
            gds = [pltpu.async_copy(tbl.at[ibuf.at[k]],
                                    gbuf.at[pl.ds(k * BLK, BLK)], sem)
                   for k in range(GB)]
            for g in gds:
                g.wait()
            pltpu.sync_copy(
                gbuf, out_hbm.at[pl.ds(pl.multiple_of(w * (GB * BLK), 8),
                                       GB * BLK)])
        return _

    pl.when((c == 0) & (s < 8))(do(tg_hbm))
    pl.when((c == 0) & (s >= 8))(do(te_hbm))
    pl.when((c == 1) & (s < 8))(do(t1_hbm))
    pl.when((c == 1) & (s >= 8))(do(t2_hbm))


def _make_bgather():
    mesh = plsc.VectorSubcoreMesh(core_axis_name="c", subcore_axis_name="s",
                                  num_cores=NC, num_subcores=NS)
    return pl.kernel(
        _bgather_body,
        out_type=jax.ShapeDtypeStruct((32 * GB * BLK, D), jnp.float32),
        mesh=mesh,
        scratch_types=[
            pltpu.VMEM((GB, BLK), jnp.int32),
            pltpu.VMEM((GB * BLK, D), jnp.float32),
            pltpu.SemaphoreType.DMA,
        ],
        compiler_params=pltpu.CompilerParams(use_tc_tiling_on_sc=False,
                                             needs_layout_passes=False),
    )


def _pack_sec(ids, nblk):
    a = ids.reshape(8, nblk, BLK)
    pad = jnp.zeros((8, GB - nblk, BLK), jnp.int32)
    return jnp.concatenate([a, pad], axis=1)


RBLK = 2944
NBLK_HALF = ACC_ROWS // RBLK


def _ssl_body(q_ref, t_ref, o_ref):
    b = pl.program_id(1)
    q = q_ref[0]
    tb = t_ref[...]
    s2 = jnp.sum(tb * tb, axis=1, keepdims=True)
    inv = 1.0 / jnp.maximum(jnp.sqrt(s2), 1e-12)
    sc = lax.dot_general(q, tb * inv, (((1,), (1,)), ((), ())),
                         preferred_element_type=jnp.float32)
    ridx = lax.broadcasted_iota(jnp.int32, (1, RBLK), 1) + b * RBLK
    z = jnp.where(ridx < U, jnp.exp(sc * (1.0 / TAU)), 0.0)
    r = jnp.sum(z, axis=1)

    @pl.when(b == 0)
    def _():
        o_ref[0, 0, :] = r

    @pl.when(b != 0)
    def _():
        o_ref[0, 0, :] = o_ref[0, 0, :] + r


def _ssl_sumexp(qn, tbl):
    out = pl.pallas_call(
        _ssl_body,
        grid=(2, NBLK_HALF),
        in_specs=[pl.BlockSpec((1, B, 16), lambda p, b: (p, 0, 0)),
                  pl.BlockSpec((RBLK, 16), lambda p, b: (p * NBLK_HALF + b, 0))],
        out_specs=pl.BlockSpec((1, 1, B), lambda p, b: (p, 0, 0)),
        out_shape=jax.ShapeDtypeStruct((2, 1, B), jnp.float32),
    )(qn, tbl)
    return out[:, 0, :]


def _normalize(x):
    return x / jnp.clip(jnp.linalg.norm(x, axis=1, keepdims=True), 1e-12, None)


def kernel(user_emb, item_emb, g_rows, g_cols, g_vals, g1_rows, g1_cols, g1_vals,
           g2_rows, g2_cols, g2_vals, user_id, item_id, neg_item_id):
    spmm = _make_spmm()
    bgather = _make_bgather()

    rp0, cp0, bnd0 = _edge_prep(g_rows, g_cols)
    rp1, cp1, bnd1 = _edge_prep(g1_rows, g1_cols)
    rp2, cp2, bnd2 = _edge_prep(g2_rows, g2_cols)

    deg_raw = spmm(jnp.ones((NP, D), jnp.float32), rp0, cp0, bnd0)[:, 0]
    deg = jnp.maximum(deg_raw, 1.0)
    invd = (1.0 / deg)[:, None]
    invd_drop = invd * (1.0 / (1.0 - DROP))

    all_emb = (jnp.zeros((NP, D), jnp.float32)
               .at[0:U].set(user_emb)
               .at[ACC_ROWS:ACC_ROWS + U].set(item_emb))
    t0 = all_emb * (deg ** -0.5)[:, None]

    def prop(rp, cp, bnd, scale):
        t1 = spmm(t0, rp, cp, bnd) * scale
        t2 = spmm(t1, rp, cp, bnd) * scale
        t3 = spmm(t2, rp, cp, bnd) * scale
        return t0 + t1 + t2 + t3

    light_g = jnp.sqrt(deg)[:, None] * prop(rp0, cp0, bnd0, invd) * 0.25
    tsum_1 = prop(rp1, cp1, bnd1, invd_drop)
    tsum_2 = prop(rp2, cp2, bnd2, invd_drop)

    iid = item_id + ACC_ROWS
    nid = neg_item_id + ACC_ROWS

    idx3d = jnp.concatenate([
        _pack_sec(jnp.concatenate([user_id, iid, nid]), 3),
        _pack_sec(jnp.concatenate([user_id, iid, nid]), 3),
        _pack_sec(jnp.concatenate([user_id, iid]), 2),
        _pack_sec(jnp.concatenate([user_id, iid]), 2),
    ], axis=0)
    rows = bgather(light_g, all_emb, tsum_1, tsum_2, idx3d)
    r4 = rows.reshape(32, GB, BLK, D)
    sec_g = r4[0:8, 0:3].reshape(3 * B, D)
    sec_e = r4[8:16, 0:3].reshape(3 * B, D)
    sec_1 = r4[16:24, 0:2].reshape(2 * B, D)
    sec_2 = r4[24:32, 0:2].reshape(2 * B, D)
    ue, pie, nie = sec_g[0:B], sec_g[B:2 * B], sec_g[2 * B:3 * B]
    ue_ego, pie_ego, nie_ego = sec_e[0:B], sec_e[B:2 * B], sec_e[2 * B:3 * B]

    pos_scores = jnp.sum(ue * pie, axis=1)
    neg_scores = jnp.sum(ue * nie, axis=1)
    bpr_loss = jnp.mean(jax.nn.softplus(neg_scores - pos_scores))
    reg_loss = (jnp.sum(ue_ego ** 2) + jnp.sum(pie_ego ** 2)
                + jnp.sum(nie_ego ** 2)) / (2.0 * B)

    ue1 = _normalize(sec_1[0:B])
    ie1 = _normalize(sec_1[B:2 * B])
    ue2 = _normalize(sec_2[0:B])
    ie2 = _normalize(sec_2[B:2 * B])
    pos_u = jnp.sum(ue1 * ue2, axis=1)
    pos_i = jnp.sum(ie1 * ie2, axis=1)

    qn = jnp.stack([ue1, ie1])
    zraw = _ssl_sumexp(qn, tsum_2)
    clog_u = jnp.log(zraw[0]) - pos_u / TAU
    clog_i = jnp.log(zraw[1]) - pos_i / TAU
    ssl_loss = jnp.sum(clog_u + clog_i)

    return bpr_loss + ssl_loss * LMBD_SSL + reg_loss * LMBD_REG


# speedup vs baseline: 1.2408x; 1.0230x over previous
import jax
import jax.numpy as jnp
from jax import lax
from jax.experimental import pallas as pl
from jax.experimental.pallas import tpu as pltpu
from jax.experimental.pallas import tpu_sc as plsc

U = 100000
I = 100000
D = 16
TAU = 0.2
LMBD_SSL = 0.1
LMBD_REG = 1e-4
DROP = 0.1
N = U + I

NC = 2
NS = 16
BLK = 128
QD = 8
GE = BLK * QD
SPAN = 6256
ACC_ROWS = NS * SPAN
TRASH = U
NP = NC * ACC_ROWS


SB = 4
SBE = SB * BLK


def _spmm_body(t_hbm, rows_hbm, cols_hbm, bnd_hbm, out_hbm,
               acc_sh, bvec, rbufA, cbufA, libufA, rbufB, cbufB, libufB,
               gbuf0, gbuf1, tibuf,
               sem_ia, sem_ib, sem_g0, sem_g1, sem_s0, sem_s1):
    c = lax.axis_index("c")
    s = lax.axis_index("s")
    lanes = lax.iota(jnp.int32, 16)
    zero16 = jnp.zeros((16,), jnp.float32)

    def zfill(i, carry):
        gbuf0[i, :] = zero16
        return carry

    lax.fori_loop(0, SBE, zfill, 0)
    for r in range(SPAN // SBE):
        pltpu.sync_copy(
            gbuf0, acc_sh.at[pl.ds(pl.multiple_of(s * SPAN + r * SBE, 8), SBE)])
    rem = SPAN % SBE
    pltpu.sync_copy(
        gbuf0.at[pl.ds(0, rem)],
        acc_sh.at[pl.ds(pl.multiple_of(s * SPAN + (SPAN // SBE) * SBE, 8), rem)])
    for p in range(BLK // 16):
        tibuf[0, pl.ds(p * 16, 16)] = jnp.full((16,), TRASH, jnp.int32)
    plsc.subcore_barrier()

    pltpu.sync_copy(bnd_hbm, bvec)
    fs = c * 16 + s

    def pick(base):
        acc = jnp.int32(0)
        for k in range(2):
            chunk = bvec[pl.ds(base + k * 16, 16)]
            acc = acc + jnp.sum(jnp.where((k * 16) + lanes == fs, chunk, 0))
        return acc

    start = pick(0)
    end = pick(32)
    ngroups = (end - start + (GE - 1)) // GE
    npairs = (ngroups + 1) // 2
    rowbase = c * U

    def crow_of(g):
        return pl.multiple_of((start + g * GE) // BLK, 8)

    def ids_issue(g, rbuf, cbuf, sem):
        cr = crow_of(g)
        pltpu.async_copy(rows_hbm.at[pl.ds(cr, QD)], rbuf, sem)
        pltpu.async_copy(cols_hbm.at[pl.ds(cr, QD)], cbuf, sem)

    def ids_drain(g, rbuf, cbuf, sem):
        cr = crow_of(g)
        pltpu.make_async_copy(rows_hbm.at[pl.ds(cr, QD)], rbuf, sem).wait()
        pltpu.make_async_copy(cols_hbm.at[pl.ds(cr, QD)], cbuf, sem).wait()

    def group(g, rbuf, cbuf, libuf):
        goff = start + g * GE
        for q in range(QD):
            for p in range(BLK // 16):
                rid = rbuf[q, pl.ds(p * 16, 16)]
                gidx = (goff + q * BLK + p * 16) + lanes
                local = rid - rowbase
                valid = (gidx < end) & (local >= 0) & (local < U)
                libuf[q, pl.ds(p * 16, 16)] = jnp.where(valid, local, TRASH)
        gds = [pltpu.async_copy(
                   t_hbm.at[cbuf.at[q]],
                   (gbuf0 if q < SB else gbuf1).at[pl.ds((q % SB) * BLK, BLK)],
                   sem_g0)
               for q in range(QD)]
        sds = []
        for q in range(QD):
            gds[q].wait()
            sds.append(pltpu.async_copy(
                (gbuf0 if q < SB else gbuf1).at[pl.ds((q % SB) * BLK, BLK)],
                acc_sh.at[libuf.at[q]], sem_s0, add=True))
        for d in sds:
            d.wait()

    ids_issue(0, rbufA, cbufA, sem_ia)

    def pair(p, carry):
        g0 = p * 2
        ids_drain(g0, rbufA, cbufA, sem_ia)
        ids_issue(g0 + 1, rbufB, cbufB, sem_ib)
        group(g0, rbufA, cbufA, libufA)
        ids_drain(g0 + 1, rbufB, cbufB, sem_ib)
        ids_issue(g0 + 2, rbufA, cbufA, sem_ia)
        group(g0 + 1, rbufB, cbufB, libufB)
        return carry

    lax.fori_loop(0, npairs, pair, 0)
    ids_drain(2 * npairs, rbufA, cbufA, sem_ia)

    plsc.subcore_barrier()
    src_off = pl.multiple_of(s * SPAN, 8)
    dst_off = pl.multiple_of(c * ACC_ROWS + s * SPAN, 8)
    pltpu.sync_copy(acc_sh.at[pl.ds(src_off, SPAN)],
                    out_hbm.at[pl.ds(dst_off, SPAN)])


def _make_spmm():
    mesh = plsc.VectorSubcoreMesh(core_axis_name="c", subcore_axis_name="s",
                                  num_cores=NC, num_subcores=NS)
    return pl.kernel(
        _spmm_body,
        out_type=jax.ShapeDtypeStruct((NP, D), jnp.float32),
        mesh=mesh,
        scratch_types=[
            pltpu.VMEM_SHARED((ACC_ROWS, D), jnp.float32),
            pltpu.VMEM((64,), jnp.int32),
            pltpu.VMEM((QD, BLK), jnp.int32),
            pltpu.VMEM((QD, BLK), jnp.int32),
            pltpu.VMEM((QD, BLK), jnp.int32),
            pltpu.VMEM((QD, BLK), jnp.int32),
            pltpu.VMEM((QD, BLK), jnp.int32),
            pltpu.VMEM((QD, BLK), jnp.int32),
            pltpu.VMEM((SBE, D), jnp.float32),
            pltpu.VMEM((SBE, D), jnp.float32),
            pltpu.VMEM((1, BLK), jnp.int32),
            pltpu.SemaphoreType.DMA,
            pltpu.SemaphoreType.DMA,
            pltpu.SemaphoreType.DMA,
            pltpu.SemaphoreType.DMA,
            pltpu.SemaphoreType.DMA,
            pltpu.SemaphoreType.DMA,
        ],
        compiler_params=pltpu.CompilerParams(use_tc_tiling_on_sc=False,
                                             needs_layout_passes=False),
    )


def _tile_bounds(lo, hi):
    sidx = jnp.arange(NS, dtype=jnp.int32)
    raw = lo + ((hi - lo) * sidx) // NS
    st = raw & ~jnp.int32(GE - 1)
    en = jnp.concatenate([st[1:], hi[None]])
    return st, en


def _edge_prep(rows, cols):
    e = rows.shape[0]
    lp = (e // GE + 4) * GE
    split = jnp.sum((rows < U).astype(jnp.int32))
    st0, en0 = _tile_bounds(jnp.int32(0), split)
    st1, en1 = _tile_bounds(split, jnp.int32(e))
    bnd = jnp.concatenate([st0, st1, en0, en1])
    rows_p = jnp.concatenate([rows, jnp.full((lp - e,), N, jnp.int32)])
    cols_adj = jnp.where(cols >= U, cols + (ACC_ROWS - U), cols)
    cols_p = jnp.concatenate([cols_adj, jnp.zeros((lp - e,), jnp.int32)])
    return (rows_p.reshape(lp // BLK, BLK), cols_p.reshape(lp // BLK, BLK),
            bnd)


B = 1024
GB = 4


def _bgather_body(tg_hbm, te_hbm, t1_hbm, t2_hbm, idx_hbm, out_hbm,
                  ibuf, gbuf, sem):
    c = lax.axis_index("c")
    s = lax.axis_index("s")
    w = c * 16 + s
    pltpu.sync_copy(idx_hbm.at[w], ibuf)

    def do(tbl):
        def _():
            gds = [pltpu.async_copy(tbl.at[ibuf.at[k]],
                                    gbuf.at[pl.ds(k * BLK, BLK)], sem)
                   for k in range(GB)]
            for g in gds:
                g.wait()
            pltpu.sync_copy(
                gbuf, out_hbm.at[pl.ds(pl.multiple_of(w * (GB * BLK), 8),
                                       GB * BLK)])
        return _

    pl.when((c == 0) & (s < 8))(do(tg_hbm))
    pl.when((c == 0) & (s >= 8))(do(te_hbm))
    pl.when((c == 1) & (s < 8))(do(t1_hbm))
    pl.when((c == 1) & (s >= 8))(do(t2_hbm))


def _make_bgather():
    mesh = plsc.VectorSubcoreMesh(core_axis_name="c", subcore_axis_name="s",
                                  num_cores=NC, num_subcores=NS)
    return pl.kernel(
        _bgather_body,
        out_type=jax.ShapeDtypeStruct((32 * GB * BLK, D), jnp.float32),
        mesh=mesh,
        scratch_types=[
            pltpu.VMEM((GB, BLK), jnp.int32),
            pltpu.VMEM((GB * BLK, D), jnp.float32),
            pltpu.SemaphoreType.DMA,
        ],
        compiler_params=pltpu.CompilerParams(use_tc_tiling_on_sc=False,
                                             needs_layout_passes=False),
    )


def _pack_sec(ids, nblk):
    a = ids.reshape(8, nblk, BLK)
    pad = jnp.zeros((8, GB - nblk, BLK), jnp.int32)
    return jnp.concatenate([a, pad], axis=1)


RBLK = 2048
NPAD = 100352
NPAD_EXTRA = NPAD - U


def _ssl_body(q_ref, t_ref, o_ref):
    b = pl.program_id(1)
    q = q_ref[0]
    tb = t_ref[0]
    s = lax.dot_general(q, tb, (((0,), (0,)), ((), ())),
                        preferred_element_type=jnp.float32)
    r = jnp.sum(jnp.exp(s * (1.0 / TAU)), axis=1)

    @pl.when(b == 0)
    def _():
        o_ref[0, 0, :] = r

    @pl.when(b != 0)
    def _():
        o_ref[0, 0, :] = o_ref[0, 0, :] + r


def _ssl_sumexp(qt, tt):
    out = pl.pallas_call(
        _ssl_body,
        grid=(2, NPAD // RBLK),
        in_specs=[pl.BlockSpec((1, 16, B), lambda p, b: (p, 0, 0)),
                  pl.BlockSpec((1, 16, RBLK), lambda p, b: (p, 0, b))],
        out_specs=pl.BlockSpec((1, 1, B), lambda p, b: (p, 0, 0)),
        out_shape=jax.ShapeDtypeStruct((2, 1, B), jnp.float32),
    )(qt, tt)
    return out[:, 0, :]


def _normalize(x):
    return x / jnp.clip(jnp.linalg.norm(x, axis=1, keepdims=True), 1e-12, None)


def kernel(user_emb, item_emb, g_rows, g_cols, g_vals, g1_rows, g1_cols, g1_vals,
           g2_rows, g2_cols, g2_vals, user_id, item_id, neg_item_id):
    spmm = _make_spmm()
    bgather = _make_bgather()

    rp0, cp0, bnd0 = _edge_prep(g_rows, g_cols)
    rp1, cp1, bnd1 = _edge_prep(g1_rows, g1_cols)
    rp2, cp2, bnd2 = _edge_prep(g2_rows, g2_cols)

    deg_raw = spmm(jnp.ones((NP, D), jnp.float32), rp0, cp0, bnd0)[:, 0]
    deg = jnp.maximum(deg_raw, 1.0)
    invd = (1.0 / deg)[:, None]
    invd_drop = invd * (1.0 / (1.0 - DROP))

    all_emb = (jnp.zeros((NP, D), jnp.float32)
               .at[0:U].set(user_emb)
               .at[ACC_ROWS:ACC_ROWS + U].set(item_emb))
    t0 = all_emb * (deg ** -0.5)[:, None]

    def prop(rp, cp, bnd, scale):
        t1 = spmm(t0, rp, cp, bnd) * scale
        t2 = spmm(t1, rp, cp, bnd) * scale
        t3 = spmm(t2, rp, cp, bnd) * scale
        return t0 + t1 + t2 + t3

    light_g = jnp.sqrt(deg)[:, None] * prop(rp0, cp0, bnd0, invd) * 0.25
    tsum_1 = prop(rp1, cp1, bnd1, invd_drop)
    tsum_2 = prop(rp2, cp2, bnd2, invd_drop)

    iid = item_id + ACC_ROWS
    nid = neg_item_id + ACC_ROWS

    idx3d = jnp.concatenate([
        _pack_sec(jnp.concatenate([user_id, iid, nid]), 3),
        _pack_sec(jnp.concatenate([user_id, iid, nid]), 3),
        _pack_sec(jnp.concatenate([user_id, iid]), 2),
        _pack_sec(jnp.concatenate([user_id, iid]), 2),
    ], axis=0)
    rows = bgather(light_g, all_emb, tsum_1, tsum_2, idx3d)
    r4 = rows.reshape(32, GB, BLK, D)
    sec_g = r4[0:8, 0:3].reshape(3 * B, D)
    sec_e = r4[8:16, 0:3].reshape(3 * B, D)
    sec_1 = r4[16:24, 0:2].reshape(2 * B, D)
    sec_2 = r4[24:32, 0:2].reshape(2 * B, D)
    ue, pie, nie = sec_g[0:B], sec_g[B:2 * B], sec_g[2 * B:3 * B]
    ue_ego, pie_ego, nie_ego = sec_e[0:B], sec_e[B:2 * B], sec_e[2 * B:3 * B]

    pos_scores = jnp.sum(ue * pie, axis=1)
    neg_scores = jnp.sum(ue * nie, axis=1)
    bpr_loss = jnp.mean(jax.nn.softplus(neg_scores - pos_scores))
    reg_loss = (jnp.sum(ue_ego ** 2) + jnp.sum(pie_ego ** 2)
                + jnp.sum(nie_ego ** 2)) / (2.0 * B)

    ue1 = _normalize(sec_1[0:B])
    ie1 = _normalize(sec_1[B:2 * B])
    ue2 = _normalize(sec_2[0:B])
    ie2 = _normalize(sec_2[B:2 * B])
    pos_u = jnp.sum(ue1 * ue2, axis=1)
    pos_i = jnp.sum(ie1 * ie2, axis=1)

    u2n = _normalize(tsum_2[:U])
    i2n = _normalize(tsum_2[ACC_ROWS:ACC_ROWS + U])
    qt = jnp.stack([ue1.T, ie1.T])
    padz = jnp.zeros((16, NPAD - U), jnp.float32)
    tt = jnp.stack([jnp.concatenate([u2n.T, padz], axis=1),
                    jnp.concatenate([i2n.T, padz], axis=1)])
    zraw = _ssl_sumexp(qt, tt) - jnp.float32(NPAD_EXTRA)
    clog_u = jnp.log(zraw[0]) - pos_u / TAU
    clog_i = jnp.log(zraw[1]) - pos_i / TAU
    ssl_loss = jnp.sum(clog_u + clog_i)

    return bpr_loss + ssl_loss * LMBD_SSL + reg_loss * LMBD_REG


# speedup vs baseline: 1.2710x; 1.0244x over previous
import jax
import jax.numpy as jnp
from jax import lax
from jax.experimental import pallas as pl
from jax.experimental.pallas import tpu as pltpu
from jax.experimental.pallas import tpu_sc as plsc

U = 100000
I = 100000
D = 16
TAU = 0.2
LMBD_SSL = 0.1
LMBD_REG = 1e-4
DROP = 0.1
N = U + I

NC = 2
NS = 16
BLK = 128
QD = 8
GE = BLK * QD
SPAN = 6256
ACC_ROWS = NS * SPAN
TRASH = U
NP = NC * ACC_ROWS


SB = 4
SBE = SB * BLK


def _spmm_body(t_hbm, rows_hbm, cols_hbm, bnd_hbm, out_hbm,
               acc_sh, bvec, rbufA, cbufA, libufA, rbufB, cbufB, libufB,
               gbuf0, gbuf1, tibuf,
               sem_ia, sem_ib, sem_g0, sem_g1, sem_s0, sem_s1):
    c = lax.axis_index("c")
    s = lax.axis_index("s")
    lanes = lax.iota(jnp.int32, 16)
    zero16 = jnp.zeros((16,), jnp.float32)

    def zfill(i, carry):
        gbuf0[i, :] = zero16
        return carry

    lax.fori_loop(0, SBE, zfill, 0)
    for r in range(SPAN // SBE):
        pltpu.sync_copy(
            gbuf0, acc_sh.at[pl.ds(pl.multiple_of(s * SPAN + r * SBE, 8), SBE)])
    rem = SPAN % SBE
    pltpu.sync_copy(
        gbuf0.at[pl.ds(0, rem)],
        acc_sh.at[pl.ds(pl.multiple_of(s * SPAN + (SPAN // SBE) * SBE, 8), rem)])
    for p in range(BLK // 16):
        tibuf[0, pl.ds(p * 16, 16)] = jnp.full((16,), TRASH, jnp.int32)
    plsc.subcore_barrier()

    pltpu.sync_copy(bnd_hbm, bvec)
    fs = c * 16 + s

    def pick(base):
        acc = jnp.int32(0)
        for k in range(2):
            chunk = bvec[pl.ds(base + k * 16, 16)]
            acc = acc + jnp.sum(jnp.where((k * 16) + lanes == fs, chunk, 0))
        return acc

    start = pick(0)
    end = pick(32)
    ngroups = (end - start + (GE - 1)) // GE
    npairs = (ngroups + 1) // 2
    rowbase = c * U

    def crow_of(g):
        return pl.multiple_of((start + g * GE) // BLK, 8)

    def ids_issue(g, rbuf, cbuf, sem):
        cr = crow_of(g)
        pltpu.async_copy(rows_hbm.at[pl.ds(cr, QD)], rbuf, sem)
        pltpu.async_copy(cols_hbm.at[pl.ds(cr, QD)], cbuf, sem)

    def ids_drain(g, rbuf, cbuf, sem):
        cr = crow_of(g)
        pltpu.make_async_copy(rows_hbm.at[pl.ds(cr, QD)], rbuf, sem).wait()
        pltpu.make_async_copy(cols_hbm.at[pl.ds(cr, QD)], cbuf, sem).wait()

    def group(g, rbuf, cbuf, libuf):
        goff = start + g * GE
        for q in range(QD):
            for p in range(BLK // 16):
                rid = rbuf[q, pl.ds(p * 16, 16)]
                gidx = (goff + q * BLK + p * 16) + lanes
                local = rid - rowbase
                valid = (gidx < end) & (local >= 0) & (local < U)
                libuf[q, pl.ds(p * 16, 16)] = jnp.where(valid, local, TRASH)
        gds = [pltpu.async_copy(
                   t_hbm.at[cbuf.at[q]],
                   (gbuf0 if q < SB else gbuf1).at[pl.ds((q % SB) * BLK, BLK)],
                   sem_g0)
               for q in range(QD)]
        sds = []
        for q in range(QD):
            gds[q].wait()
            sds.append(pltpu.async_copy(
                (gbuf0 if q < SB else gbuf1).at[pl.ds((q % SB) * BLK, BLK)],
                acc_sh.at[libuf.at[q]], sem_s0, add=True))
        for d in sds:
            d.wait()

    ids_issue(0, rbufA, cbufA, sem_ia)

    def pair(p, carry):
        g0 = p * 2
        ids_drain(g0, rbufA, cbufA, sem_ia)
        ids_issue(g0 + 1, rbufB, cbufB, sem_ib)
        group(g0, rbufA, cbufA, libufA)
        ids_drain(g0 + 1, rbufB, cbufB, sem_ib)
        ids_issue(g0 + 2, rbufA, cbufA, sem_ia)
        group(g0 + 1, rbufB, cbufB, libufB)
        return carry

    lax.fori_loop(0, npairs, pair, 0)
    ids_drain(2 * npairs, rbufA, cbufA, sem_ia)

    plsc.subcore_barrier()
    src_off = pl.multiple_of(s * SPAN, 8)
    dst_off = pl.multiple_of(c * ACC_ROWS + s * SPAN, 8)
    pltpu.sync_copy(acc_sh.at[pl.ds(src_off, SPAN)],
                    out_hbm.at[pl.ds(dst_off, SPAN)])


def _make_spmm():
    mesh = plsc.VectorSubcoreMesh(core_axis_name="c", subcore_axis_name="s",
                                  num_cores=NC, num_subcores=NS)
    return pl.kernel(
        _spmm_body,
        out_type=jax.ShapeDtypeStruct((NP, D), jnp.float32),
        mesh=mesh,
        scratch_types=[
            pltpu.VMEM_SHARED((ACC_ROWS, D), jnp.float32),
            pltpu.VMEM((64,), jnp.int32),
            pltpu.VMEM((QD, BLK), jnp.int32),
            pltpu.VMEM((QD, BLK), jnp.int32),
            pltpu.VMEM((QD, BLK), jnp.int32),
            pltpu.VMEM((QD, BLK), jnp.int32),
            pltpu.VMEM((QD, BLK), jnp.int32),
            pltpu.VMEM((QD, BLK), jnp.int32),
            pltpu.VMEM((SBE, D), jnp.float32),
            pltpu.VMEM((SBE, D), jnp.float32),
            pltpu.VMEM((1, BLK), jnp.int32),
            pltpu.SemaphoreType.DMA,
            pltpu.SemaphoreType.DMA,
            pltpu.SemaphoreType.DMA,
            pltpu.SemaphoreType.DMA,
            pltpu.SemaphoreType.DMA,
            pltpu.SemaphoreType.DMA,
        ],
        compiler_params=pltpu.CompilerParams(use_tc_tiling_on_sc=False,
                                             needs_layout_passes=False),
    )


def _tile_bounds(lo, hi):
    sidx = jnp.arange(NS, dtype=jnp.int32)
    raw = lo + ((hi - lo) * sidx) // NS
    st = raw & ~jnp.int32(GE - 1)
    en = jnp.concatenate([st[1:], hi[None]])
    return st, en


def _edge_prep(rows, cols):
    e = rows.shape[0]
    lp = (e // GE + 4) * GE
    split = jnp.sum((rows < U).astype(jnp.int32))
    st0, en0 = _tile_bounds(jnp.int32(0), split)
    st1, en1 = _tile_bounds(split, jnp.int32(e))
    bnd = jnp.concatenate([st0, st1, en0, en1])
    rows_p = jnp.concatenate([rows, jnp.full((lp - e,), N, jnp.int32)])
    cols_adj = jnp.where(cols >= U, cols + (ACC_ROWS - U), cols)
    cols_p = jnp.concatenate([cols_adj, jnp.zeros((lp - e,), jnp.int32)])
    return (rows_p.reshape(lp // BLK, BLK), cols_p.reshape(lp // BLK, BLK),
            bnd)


B = 1024

RBLK = 2048
NPAD = 100352
NPAD_EXTRA = NPAD - U


def _ssl_body(q_ref, t_ref, o_ref):
    b = pl.program_id(1)
    q = q_ref[0]
    tb = t_ref[0]
    s = lax.dot_general(q, tb, (((0,), (0,)), ((), ())),
                        preferred_element_type=jnp.float32)
    r = jnp.sum(jnp.exp(s * (1.0 / TAU)), axis=1)

    @pl.when(b == 0)
    def _():
        o_ref[0, 0, :] = r

    @pl.when(b != 0)
    def _():
        o_ref[0, 0, :] = o_ref[0, 0, :] + r


def _ssl_sumexp(qt, tt):
    out = pl.pallas_call(
        _ssl_body,
        grid=(2, NPAD // RBLK),
        in_specs=[pl.BlockSpec((1, 16, B), lambda p, b: (p, 0, 0)),
                  pl.BlockSpec((1, 16, RBLK), lambda p, b: (p, 0, b))],
        out_specs=pl.BlockSpec((1, 1, B), lambda p, b: (p, 0, 0)),
        out_shape=jax.ShapeDtypeStruct((2, 1, B), jnp.float32),
    )(qt, tt)
    return out[:, 0, :]


def _normalize(x):
    return x / jnp.clip(jnp.linalg.norm(x, axis=1, keepdims=True), 1e-12, None)


def kernel(user_emb, item_emb, g_rows, g_cols, g_vals, g1_rows, g1_cols, g1_vals,
           g2_rows, g2_cols, g2_vals, user_id, item_id, neg_item_id):
    spmm = _make_spmm()

    rp0, cp0, bnd0 = _edge_prep(g_rows, g_cols)
    rp1, cp1, bnd1 = _edge_prep(g1_rows, g1_cols)
    rp2, cp2, bnd2 = _edge_prep(g2_rows, g2_cols)

    deg_raw = spmm(jnp.ones((NP, D), jnp.float32), rp0, cp0, bnd0)[:, 0]
    deg = jnp.maximum(deg_raw, 1.0)
    invd = (1.0 / deg)[:, None]
    invd_drop = invd * (1.0 / (1.0 - DROP))

    all_emb = (jnp.zeros((NP, D), jnp.float32)
               .at[0:U].set(user_emb)
               .at[ACC_ROWS:ACC_ROWS + U].set(item_emb))
    t0 = all_emb * (deg ** -0.5)[:, None]

    def prop(rp, cp, bnd, scale):
        t1 = spmm(t0, rp, cp, bnd) * scale
        t2 = spmm(t1, rp, cp, bnd) * scale
        t3 = spmm(t2, rp, cp, bnd) * scale
        return t0 + t1 + t2 + t3

    light_g = jnp.sqrt(deg)[:, None] * prop(rp0, cp0, bnd0, invd) * 0.25
    tsum_1 = prop(rp1, cp1, bnd1, invd_drop)
    tsum_2 = prop(rp2, cp2, bnd2, invd_drop)

    iid = item_id + ACC_ROWS
    nid = neg_item_id + ACC_ROWS
    ue = light_g[user_id]
    pie = light_g[iid]
    nie = light_g[nid]
    ue_ego = all_emb[user_id]
    pie_ego = all_emb[iid]
    nie_ego = all_emb[nid]

    pos_scores = jnp.sum(ue * pie, axis=1)
    neg_scores = jnp.sum(ue * nie, axis=1)
    bpr_loss = jnp.mean(jax.nn.softplus(neg_scores - pos_scores))
    reg_loss = (jnp.sum(ue_ego ** 2) + jnp.sum(pie_ego ** 2)
                + jnp.sum(nie_ego ** 2)) / (2.0 * B)

    u2n = _normalize(tsum_2[:U])
    i2n = _normalize(tsum_2[ACC_ROWS:ACC_ROWS + U])
    ue1 = _normalize(tsum_1[user_id])
    ie1 = _normalize(tsum_1[iid])
    ue2 = u2n[user_id]
    ie2 = i2n[item_id]
    pos_u = jnp.sum(ue1 * ue2, axis=1)
    pos_i = jnp.sum(ie1 * ie2, axis=1)
    qt = jnp.stack([ue1.T, ie1.T])
    padz = jnp.zeros((16, NPAD - U), jnp.float32)
    tt = jnp.stack([jnp.concatenate([u2n.T, padz], axis=1),
                    jnp.concatenate([i2n.T, padz], axis=1)])
    zraw = _ssl_sumexp(qt, tt) - jnp.float32(NPAD_EXTRA)
    clog_u = jnp.log(zraw[0]) - pos_u / TAU
    clog_i = jnp.log(zraw[1]) - pos_i / TAU
    ssl_loss = jnp.sum(clog_u + clog_i)

    return bpr_loss + ssl_loss * LMBD_SSL + reg_loss * LMBD_REG


# speedup vs baseline: 1.2736x; 1.0020x over previous
import jax
import jax.numpy as jnp
from jax import lax
from jax.experimental import pallas as pl
from jax.experimental.pallas import tpu as pltpu
from jax.experimental.pallas import tpu_sc as plsc

U = 100000
I = 100000
D = 16
TAU = 0.2
LMBD_SSL = 0.1
LMBD_REG = 1e-4
DROP = 0.1
N = U + I

NC = 2
NS = 16
BLK = 128
QD = 8
GE = BLK * QD
SPAN = 6256
ACC_ROWS = NS * SPAN
TRASH = U
NP = NC * ACC_ROWS


SB = 4
SBE = SB * BLK


def _spmm_body(t_hbm, rows_hbm, cols_hbm, bnd_hbm, out_hbm,
               acc_sh, bvec, rbufA, cbufA, libufA, rbufB, cbufB, libufB,
               gbuf0, gbuf1, sem_ia, sem_ib, sem_g0, sem_s0):
    c = lax.axis_index("c")
    s = lax.axis_index("s")
    lanes = lax.iota(jnp.int32, 16)
    zero16 = jnp.zeros((16,), jnp.float32)

    def zfill(i, carry):
        gbuf0[i, :] = zero16
        return carry

    lax.fori_loop(0, SBE, zfill, 0)
    for r in range(SPAN // SBE):
        pltpu.sync_copy(
            gbuf0, acc_sh.at[pl.ds(pl.multiple_of(s * SPAN + r * SBE, 8), SBE)])
    rem = SPAN % SBE
    pltpu.sync_copy(
        gbuf0.at[pl.ds(0, rem)],
        acc_sh.at[pl.ds(pl.multiple_of(s * SPAN + (SPAN // SBE) * SBE, 8), rem)])
    plsc.subcore_barrier()

    pltpu.sync_copy(bnd_hbm, bvec)
    fs = c * 16 + s

    def pick(base):
        acc = jnp.int32(0)
        for k in range(2):
            chunk = bvec[pl.ds(base + k * 16, 16)]
            acc = acc + jnp.sum(jnp.where((k * 16) + lanes == fs, chunk, 0))
        return acc

    start = pick(0)
    end = pick(32)
    ngroups = (end - start + (GE - 1)) // GE
    npairs = (ngroups + 1) // 2
    rowbase = c * U

    def crow_of(g):
        return pl.multiple_of((start + g * GE) // BLK, 8)

    def ids_issue(g, rbuf, cbuf, sem):
        cr = crow_of(g)
        pltpu.async_copy(rows_hbm.at[pl.ds(cr, QD)], rbuf, sem)
        pltpu.async_copy(cols_hbm.at[pl.ds(cr, QD)], cbuf, sem)

    def ids_drain(g, rbuf, cbuf, sem):
        cr = crow_of(g)
        pltpu.make_async_copy(rows_hbm.at[pl.ds(cr, QD)], rbuf, sem).wait()
        pltpu.make_async_copy(cols_hbm.at[pl.ds(cr, QD)], cbuf, sem).wait()

    def group(g, rbuf, cbuf, libuf):
        goff = start + g * GE
        for q in range(QD):
            for p in range(BLK // 16):
                rid = rbuf[q, pl.ds(p * 16, 16)]
                gidx = (goff + q * BLK + p * 16) + lanes
                local = rid - rowbase
                valid = (gidx < end) & (local >= 0) & (local < U)
                libuf[q, pl.ds(p * 16, 16)] = jnp.where(valid, local, TRASH)
        gds = [pltpu.async_copy(
                   t_hbm.at[cbuf.at[q]],
                   (gbuf0 if q < SB else gbuf1).at[pl.ds((q % SB) * BLK, BLK)],
                   sem_g0)
               for q in range(QD)]
        sds = []
        for q in range(QD):
            gds[q].wait()
            sds.append(pltpu.async_copy(
                (gbuf0 if q < SB else gbuf1).at[pl.ds((q % SB) * BLK, BLK)],
                acc_sh.at[libuf.at[q]], sem_s0, add=True))
        for d in sds:
            d.wait()

    ids_issue(0, rbufA, cbufA, sem_ia)

    def pair(p, carry):
        g0 = p * 2
        ids_drain(g0, rbufA, cbufA, sem_ia)
        ids_issue(g0 + 1, rbufB, cbufB, sem_ib)
        group(g0, rbufA, cbufA, libufA)
        ids_drain(g0 + 1, rbufB, cbufB, sem_ib)
        ids_issue(g0 + 2, rbufA, cbufA, sem_ia)
        group(g0 + 1, rbufB, cbufB, libufB)
        return carry

    lax.fori_loop(0, npairs, pair, 0)
    ids_drain(2 * npairs, rbufA, cbufA, sem_ia)

    plsc.subcore_barrier()
    src_off = pl.multiple_of(s * SPAN, 8)
    dst_off = pl.multiple_of(c * ACC_ROWS + s * SPAN, 8)
    pltpu.sync_copy(acc_sh.at[pl.ds(src_off, SPAN)],
                    out_hbm.at[pl.ds(dst_off, SPAN)])


def _make_spmm():
    mesh = plsc.VectorSubcoreMesh(core_axis_name="c", subcore_axis_name="s",
                                  num_cores=NC, num_subcores=NS)
    return pl.kernel(
        _spmm_body,
        out_type=jax.ShapeDtypeStruct((NP, D), jnp.float32),
        mesh=mesh,
        scratch_types=[
            pltpu.VMEM_SHARED((ACC_ROWS, D), jnp.float32),
            pltpu.VMEM((64,), jnp.int32),
            pltpu.VMEM((QD, BLK), jnp.int32),
            pltpu.VMEM((QD, BLK), jnp.int32),
            pltpu.VMEM((QD, BLK), jnp.int32),
            pltpu.VMEM((QD, BLK), jnp.int32),
            pltpu.VMEM((QD, BLK), jnp.int32),
            pltpu.VMEM((QD, BLK), jnp.int32),
            pltpu.VMEM((SBE, D), jnp.float32),
            pltpu.VMEM((SBE, D), jnp.float32),
            pltpu.SemaphoreType.DMA,
            pltpu.SemaphoreType.DMA,
            pltpu.SemaphoreType.DMA,
            pltpu.SemaphoreType.DMA,
        ],
        compiler_params=pltpu.CompilerParams(use_tc_tiling_on_sc=False,
                                             needs_layout_passes=False),
    )


def _tile_bounds(lo, hi):
    sidx = jnp.arange(NS, dtype=jnp.int32)
    raw = lo + ((hi - lo) * sidx) // NS
    st = raw & ~jnp.int32(GE - 1)
    en = jnp.concatenate([st[1:], hi[None]])
    return st, en


def _edge_prep(rows, cols):
    e = rows.shape[0]
    lp = (e // GE + 4) * GE
    split = jnp.sum((rows < U).astype(jnp.int32))
    st0, en0 = _tile_bounds(jnp.int32(0), split)
    st1, en1 = _tile_bounds(split, jnp.int32(e))
    bnd = jnp.concatenate([st0, st1, en0, en1])
    rows_p = jnp.concatenate([rows, jnp.full((lp - e,), N, jnp.int32)])
    cols_adj = jnp.where(cols >= U, cols + (ACC_ROWS - U), cols)
    cols_p = jnp.concatenate([cols_adj, jnp.zeros((lp - e,), jnp.int32)])
    return (rows_p.reshape(lp // BLK, BLK), cols_p.reshape(lp // BLK, BLK),
            bnd)


B = 1024

RBLK = 2048
NPAD = 100352
NPAD_EXTRA = NPAD - U


def _ssl_body(q_ref, t_ref, o_ref):
    b = pl.program_id(1)
    q = q_ref[0]
    tb = t_ref[0]
    s = lax.dot_general(q, tb, (((0,), (0,)), ((), ())),
                        preferred_element_type=jnp.float32)
    r = jnp.sum(jnp.exp(s * (1.0 / TAU)), axis=1)

    @pl.when(b == 0)
    def _():
        o_ref[0, 0, :] = r

    @pl.when(b != 0)
    def _():
        o_ref[0, 0, :] = o_ref[0, 0, :] + r


def _ssl_sumexp(qt, tt):
    out = pl.pallas_call(
        _ssl_body,
        grid=(2, NPAD // RBLK),
        in_specs=[pl.BlockSpec((1, 16, B), lambda p, b: (p, 0, 0)),
                  pl.BlockSpec((1, 16, RBLK), lambda p, b: (p, 0, b))],
        out_specs=pl.BlockSpec((1, 1, B), lambda p, b: (p, 0, 0)),
        out_shape=jax.ShapeDtypeStruct((2, 1, B), jnp.float32),
    )(qt, tt)
    return out[:, 0, :]


def _normalize(x):
    return x / jnp.clip(jnp.linalg.norm(x, axis=1, keepdims=True), 1e-12, None)


def kernel(user_emb, item_emb, g_rows, g_cols, g_vals, g1_rows, g1_cols, g1_vals,
           g2_rows, g2_cols, g2_vals, user_id, item_id, neg_item_id):
    spmm = _make_spmm()

    rp0, cp0, bnd0 = _edge_prep(g_rows, g_cols)
    rp1, cp1, bnd1 = _edge_prep(g1_rows, g1_cols)
    rp2, cp2, bnd2 = _edge_prep(g2_rows, g2_cols)

    deg_raw = spmm(jnp.ones((NP, D), jnp.float32), rp0, cp0, bnd0)[:, 0]
    deg = jnp.maximum(deg_raw, 1.0)
    invd = (1.0 / deg)[:, None]
    invd_drop = invd * (1.0 / (1.0 - DROP))

    all_emb = (jnp.zeros((NP, D), jnp.float32)
               .at[0:U].set(user_emb)
               .at[ACC_ROWS:ACC_ROWS + U].set(item_emb))
    t0 = all_emb * (deg ** -0.5)[:, None]

    def prop(rp, cp, bnd, scale):
        t1 = spmm(t0, rp, cp, bnd) * scale
        t2 = spmm(t1, rp, cp, bnd) * scale
        t3 = spmm(t2, rp, cp, bnd) * scale
        return t0 + t1 + t2 + t3

    light_g = jnp.sqrt(deg)[:, None] * prop(rp0, cp0, bnd0, invd) * 0.25
    tsum_1 = prop(rp1, cp1, bnd1, invd_drop)
    tsum_2 = prop(rp2, cp2, bnd2, invd_drop)

    iid = item_id + ACC_ROWS
    nid = neg_item_id + ACC_ROWS
    ue = light_g[user_id]
    pie = light_g[iid]
    nie = light_g[nid]
    ue_ego = all_emb[user_id]
    pie_ego = all_emb[iid]
    nie_ego = all_emb[nid]

    pos_scores = jnp.sum(ue * pie, axis=1)
    neg_scores = jnp.sum(ue * nie, axis=1)
    bpr_loss = jnp.mean(jax.nn.softplus(neg_scores - pos_scores))
    reg_loss = (jnp.sum(ue_ego ** 2) + jnp.sum(pie_ego ** 2)
                + jnp.sum(nie_ego ** 2)) / (2.0 * B)

    u2n = _normalize(tsum_2[:U])
    i2n = _normalize(tsum_2[ACC_ROWS:ACC_ROWS + U])
    ue1 = _normalize(tsum_1[user_id])
    ie1 = _normalize(tsum_1[iid])
    ue2 = u2n[user_id]
    ie2 = i2n[item_id]
    pos_u = jnp.sum(ue1 * ue2, axis=1)
    pos_i = jnp.sum(ie1 * ie2, axis=1)
    qt = jnp.stack([ue1.T, ie1.T])
    padz = jnp.zeros((16, NPAD - U), jnp.float32)
    tt = jnp.stack([jnp.concatenate([u2n.T, padz], axis=1),
                    jnp.concatenate([i2n.T, padz], axis=1)])
    zraw = _ssl_sumexp(qt, tt) - jnp.float32(NPAD_EXTRA)
    clog_u = jnp.log(zraw[0]) - pos_u / TAU
    clog_i = jnp.log(zraw[1]) - pos_i / TAU
    ssl_loss = jnp.sum(clog_u + clog_i)

    return bpr_loss + ssl_loss * LMBD_SSL + reg_loss * LMBD_REG
